# baseline (speedup 1.0000x reference)
import functools

import jax
import jax.numpy as jnp
import numpy as np
from jax import lax
from jax.experimental import pallas as pl
from jax.experimental.pallas import tpu as pltpu
from jax.experimental.pallas import tpu_sc as plsc

N_TOK = 2048
D_MODEL = 1024
D_SAE = 16384
K = 32

AG = 32
NB = D_SAE // AG
R2 = 64
NW = 32
TPW = N_TOK // NW


def _k1_body(x_ref, w_ref, b_ref, out_ref):
    acc = jnp.dot(x_ref[...], w_ref[...], preferred_element_type=jnp.float32)
    out_ref[0] = acc + b_ref[0]


def _encode(x, W_enc, b_enc, interpret=False):
    b2 = b_enc.reshape(AG, 1, NB)
    return pl.pallas_call(
        _k1_body,
        grid=(AG,),
        in_specs=[
            pl.BlockSpec((N_TOK, D_MODEL), lambda a: (0, 0)),
            pl.BlockSpec((D_MODEL, NB), lambda a: (0, a)),
            pl.BlockSpec((1, 1, NB), lambda a: (a, 0, 0)),
        ],
        out_specs=pl.BlockSpec((1, N_TOK, NB), lambda a: (a, 0, 0)),
        out_shape=jax.ShapeDtypeStruct((AG, N_TOK, NB), jnp.float32),
        interpret=interpret,
    )(x, W_enc, b2)


def _xor_shuffle(x, d):
    parts = []
    for s in range(0, AG, 2 * d):
        parts.append(x[s + d:s + 2 * d])
        parts.append(x[s:s + d])
    return jnp.concatenate(parts, axis=0)


def _rev0(x):
    return jnp.concatenate([x[AG - 1 - j:AG - j] for j in range(AG)], axis=0)


def _gt(av, ai, bv, bi):
    return (av > bv) | ((av == bv) & (ai < bi))


def _cmpex(v, i, d, take_self):
    pv = _xor_shuffle(v, d)
    pi = _xor_shuffle(i, d)
    g = _gt(v, i, pv, pi)
    ts = ~(g ^ take_self)
    return jnp.where(ts, v, pv), jnp.where(ts, i, pi)


def _sort32_desc(v, i, a):
    k = 2
    while k <= AG:
        d = k // 2
        while d >= 1:
            ts = ((a & d) == 0) == ((a & k) == 0)
            v, i = _cmpex(v, i, d, ts)
            d //= 2
        k *= 2
    return v, i


def _merge_desc(v, i, a):
    h = v.shape[-1] // 2
    av, ai = v[..., :h], i[..., :h]
    bv, bi = _rev0(v[..., h:]), _rev0(i[..., h:])
    g = _gt(av, ai, bv, bi)
    v = jnp.where(g, av, bv)
    i = jnp.where(g, ai, bi)
    d = AG // 2
    while d >= 1:
        v, i = _cmpex(v, i, d, (a & d) == 0)
        d //= 2
    return v, i


def _k2_body(pre_ref, acts_ref, tv_ref, ti_ref, lsp_ref):
    rb = pl.program_id(0)
    v = pre_ref[...]
    i = (lax.broadcasted_iota(jnp.int16, v.shape, 0) * jnp.int16(NB)
         + lax.broadcasted_iota(jnp.int16, v.shape, 2))
    a = lax.broadcasted_iota(jnp.int32, (AG, 1, 1), 0)
    v, i = _sort32_desc(v, i, a)
    while v.shape[-1] > 1:
        v, i = _merge_desc(v, i, a)
    tv_ref[...] = v
    ti_ref[...] = i.astype(jnp.int32)
    thr_v = v[AG - 1]
    thr_i = i[AG - 1]
    for a in range(AG):
        pv = pre_ref[a]
        ia = (lax.broadcasted_iota(jnp.int16, pv.shape, 1)
              + jnp.int16(a * NB))
        keep = (pv > thr_v) | ((pv == thr_v) & (ia <= thr_i))
        acts_ref[:, a * NB:(a + 1) * NB] = jnp.where(
            keep, jnp.maximum(pv, 0.0), 0.0)

    @pl.when(rb == 0)
    def _():
        lsp_ref[...] = jnp.zeros((1, 1), jnp.float32)

    lsp_ref[...] += jnp.sum(jnp.maximum(v, 0.0)).reshape(1, 1)

    @pl.when(rb == pl.num_programs(0) - 1)
    def _():
        lsp_ref[...] = lsp_ref[...] / (N_TOK * D_SAE)


def _topk_acts(pre3d, interpret=False):
    nblk = N_TOK // R2
    return pl.pallas_call(
        _k2_body,
        grid=(nblk,),
        in_specs=[pl.BlockSpec((AG, R2, NB), lambda r: (0, r, 0))],
        out_specs=[
            pl.BlockSpec((R2, D_SAE), lambda r: (r, 0)),
            pl.BlockSpec((AG, R2, 1), lambda r: (0, r, 0)),
            pl.BlockSpec((AG, R2, 1), lambda r: (0, r, 0)),
            pl.BlockSpec((1, 1), lambda r: (0, 0)),
        ],
        out_shape=[
            jax.ShapeDtypeStruct((N_TOK, D_SAE), jnp.float32),
            jax.ShapeDtypeStruct((AG, N_TOK, 1), jnp.float32),
            jax.ShapeDtypeStruct((AG, N_TOK, 1), jnp.int32),
            jax.ShapeDtypeStruct((1, 1), jnp.float32),
        ],
        interpret=interpret,
    )(pre3d)


def _k3_body(idx_hbm, vals_hbm, wdec_hbm, bdec_hbm, out_hbm,
             idx_v, vals_v, bdec_v, bvals_v, rows_v, acc_v, sem0, sem1, osem):
    wid = lax.axis_index("s") * 2 + lax.axis_index("c")
    base = wid * TPW
    pltpu.sync_copy(idx_hbm.at[pl.ds(base * K, TPW * K)], idx_v)
    pltpu.sync_copy(bdec_hbm, bdec_v)
    sems = (sem0, sem1)

    def gather(t, buf, sem):
        start = pl.multiple_of(t * K, 8)
        pltpu.make_async_copy(
            wdec_hbm.at[idx_v.at[pl.ds(start, K)]], rows_v.at[buf],
            sem).start()

    gather(0, 0, sem0)

    def pair_body(p, _):
        for b in range(2):
            t = p * 2 + b
            nb = 1 - b

            @pl.when(t + 1 < TPW)
            def _():
                gather(t + 1, nb, sems[nb])

            pltpu.sync_copy(vals_hbm.at[base + t], vals_v)
            for k in range(K):
                bvals_v[k] = jnp.maximum(vals_v[pl.ds(k * 16, 16)], 0.0)
            pltpu.make_async_copy(
                wdec_hbm.at[idx_v.at[pl.ds(pl.multiple_of(t * K, 8), K)]],
                rows_v.at[b], sems[b]).wait()

            @pl.when(t >= 2)
            def _():
                pltpu.make_async_copy(
                    acc_v.at[b], out_hbm.at[base + t - 2], osem).wait()

            def cb(c, _):
                sl = pl.ds(pl.multiple_of(c * 16, 16), 16)

                def kb(kk, acc):
                    a0 = acc
                    for j in range(4):
                        a0 = a0 + bvals_v[kk * 4 + j] * rows_v[b, kk * 4 + j, sl]
                    return a0

                acc_v[b, sl] = lax.fori_loop(0, K // 4, kb, bdec_v[sl])
                return 0

            lax.fori_loop(0, D_MODEL // 16, cb, 0)
            pltpu.make_async_copy(
                acc_v.at[b], out_hbm.at[base + t], osem).start()
        return 0

    lax.fori_loop(0, TPW // 2, pair_body, 0)
    for b in range(2):
        pltpu.make_async_copy(
            acc_v.at[b], out_hbm.at[base + TPW - 2 + b], osem).wait()


def _decode_sc(idx2d, vals2d, W_dec, b_dec):
    mesh = plsc.VectorSubcoreMesh(core_axis_name="c", subcore_axis_name="s")
    f = functools.partial(
        pl.kernel,
        mesh=mesh,
        out_type=jax.ShapeDtypeStruct((N_TOK, D_MODEL), jnp.float32),
        scratch_types=[
            pltpu.VMEM((TPW * K,), jnp.int32),
            pltpu.VMEM((K * 16,), jnp.float32),
            pltpu.VMEM((D_MODEL,), jnp.float32),
            pltpu.VMEM((K, 16), jnp.float32),
            pltpu.VMEM((2, K, D_MODEL), jnp.float32),
            pltpu.VMEM((2, D_MODEL), jnp.float32),
            pltpu.SemaphoreType.DMA,
            pltpu.SemaphoreType.DMA,
            pltpu.SemaphoreType.DMA,
        ],
    )(_k3_body)
    vals_b = jnp.broadcast_to(
        vals2d[:, :, None], (N_TOK, K, 16)).reshape(N_TOK, K * 16)
    return f(idx2d.reshape(-1), vals_b, W_dec, b_dec)


def _k4_body(x_ref, xr_ref, out_ref):
    rb = pl.program_id(0)

    @pl.when(rb == 0)
    def _():
        out_ref[...] = jnp.zeros((1, 1), jnp.float32)

    d = x_ref[...] - xr_ref[...]
    out_ref[...] += jnp.sum(d * d).reshape(1, 1)

    @pl.when(rb == pl.num_programs(0) - 1)
    def _():
        out_ref[...] = out_ref[...] / (N_TOK * D_MODEL)


def _loss_recon(x, x_rec, interpret=False):
    nblk = 8
    rb = N_TOK // nblk
    return pl.pallas_call(
        _k4_body,
        grid=(nblk,),
        in_specs=[
            pl.BlockSpec((rb, D_MODEL), lambda r: (r, 0)),
            pl.BlockSpec((rb, D_MODEL), lambda r: (r, 0)),
        ],
        out_specs=pl.BlockSpec((1, 1), lambda r: (0, 0)),
        out_shape=jax.ShapeDtypeStruct((1, 1), jnp.float32),
        interpret=interpret,
    )(x, x_rec)


def kernel(x, W_enc, b_enc, W_dec, b_dec):
    pre3d = _encode(x, W_enc, b_enc)
    acts, tv3, ti3, lsp = _topk_acts(pre3d)
    topk_values = tv3.reshape(AG, N_TOK).T
    topk_indices = ti3.reshape(AG, N_TOK).T
    x_rec = _decode_sc(topk_indices, topk_values, W_dec, b_dec)
    lrec = _loss_recon(x, x_rec)
    return (x_rec, acts, topk_indices,
            lrec.reshape(()), lsp.reshape(()))


# speedup vs baseline: 1.6134x; 1.6134x over previous
import functools

import jax
import jax.numpy as jnp
from jax import lax
from jax.experimental import pallas as pl
from jax.experimental.pallas import tpu as pltpu
from jax.experimental.pallas import tpu_sc as plsc

N_TOK = 2048
D_MODEL = 1024
D_SAE = 16384
K = 32

AG = 32
NB = D_SAE // AG
R2 = 128
NW = 32
TPW = N_TOK // NW


def _k1_body(x_ref, w_ref, b_ref, out_ref):
    acc = jnp.dot(x_ref[...], w_ref[...], preferred_element_type=jnp.float32)
    out_ref[0] = acc + b_ref[0]


def _encode(x, W_enc, b_enc, interpret=False):
    b2 = b_enc.reshape(AG, 1, NB)
    return pl.pallas_call(
        _k1_body,
        grid=(AG,),
        in_specs=[
            pl.BlockSpec((N_TOK, D_MODEL), lambda a: (0, 0)),
            pl.BlockSpec((D_MODEL, NB), lambda a: (0, a)),
            pl.BlockSpec((1, 1, NB), lambda a: (a, 0, 0)),
        ],
        out_specs=pl.BlockSpec((1, N_TOK, NB), lambda a: (a, 0, 0)),
        out_shape=jax.ShapeDtypeStruct((AG, N_TOK, NB), jnp.float32),
        interpret=interpret,
    )(x, W_enc, b2)


def _xor_shuffle(x, d):
    parts = []
    for s in range(0, AG, 2 * d):
        parts.append(x[s + d:s + 2 * d])
        parts.append(x[s:s + d])
    return jnp.concatenate(parts, axis=0)


def _rev0(x):
    return jnp.concatenate([x[AG - 1 - j:AG - j] for j in range(AG)], axis=0)


def _gt(av, ai, bv, bi):
    return (av > bv) | ((av == bv) & (ai < bi))


def _cmpex(v, i, d, ts_np):
    pv = _xor_shuffle(v, d)
    pi = _xor_shuffle(i, d)
    g = _gt(v, i, pv, pi)
    runs = []
    s = 0
    for a in range(1, AG + 1):
        if a == AG or ts_np[a] != ts_np[s]:
            runs.append((s, a, ts_np[s]))
            s = a
    ov, oi = [], []
    for s, e, take in runs:
        gs = g[s:e]
        if take:
            ov.append(jnp.where(gs, v[s:e], pv[s:e]))
            oi.append(jnp.where(gs, i[s:e], pi[s:e]))
        else:
            ov.append(jnp.where(gs, pv[s:e], v[s:e]))
            oi.append(jnp.where(gs, pi[s:e], i[s:e]))
    return jnp.concatenate(ov, axis=0), jnp.concatenate(oi, axis=0)


def _sort32_desc(v, i):
    k = 2
    while k <= AG:
        d = k // 2
        while d >= 1:
            ts = [((a & d) == 0) == ((a & k) == 0) for a in range(AG)]
            v, i = _cmpex(v, i, d, ts)
            d //= 2
        k *= 2
    return v, i


def _merge_desc(v, i):
    h = v.shape[-1] // 2
    av, ai = v[..., :h], i[..., :h]
    bv, bi = _rev0(v[..., h:]), _rev0(i[..., h:])
    g = _gt(av, ai, bv, bi)
    v = jnp.where(g, av, bv)
    i = jnp.where(g, ai, bi)
    d = AG // 2
    while d >= 1:
        v, i = _cmpex(v, i, d, [(a & d) == 0 for a in range(AG)])
        d //= 2
    return v, i


def _k2_body(pre_ref, acts_ref, tv_ref, ti_ref, vb_ref, lsp_ref):
    rb = pl.program_id(0)
    v = pre_ref[...]
    i = (lax.broadcasted_iota(jnp.int32, v.shape, 0) * NB
         + lax.broadcasted_iota(jnp.int32, v.shape, 2))
    v, i = _sort32_desc(v, i)
    while v.shape[-1] > 1:
        v, i = _merge_desc(v, i)
    tv_ref[...] = jnp.concatenate([v[k] for k in range(K)], axis=1)
    ti_ref[...] = jnp.concatenate([i[k] for k in range(K)], axis=1)
    vb_ref[...] = jnp.concatenate(
        [jnp.broadcast_to(v[k], (R2, 16)) for k in range(K)], axis=1)
    thr_v = v[AG - 1]
    thr_i = i[AG - 1]
    for a in range(AG):
        pv = pre_ref[a]
        ia = lax.broadcasted_iota(jnp.int32, pv.shape, 1) + a * NB
        keep = (pv > thr_v) | ((pv == thr_v) & (ia <= thr_i))
        acts_ref[:, a * NB:(a + 1) * NB] = jnp.where(
            keep, jnp.maximum(pv, 0.0), 0.0)

    @pl.when(rb == 0)
    def _():
        lsp_ref[...] = jnp.zeros((1, 1), jnp.float32)

    lsp_ref[...] += jnp.sum(jnp.maximum(v, 0.0)).reshape(1, 1)

    @pl.when(rb == pl.num_programs(0) - 1)
    def _():
        lsp_ref[...] = lsp_ref[...] / (N_TOK * D_SAE)


def _topk_acts(pre3d, interpret=False):
    nblk = N_TOK // R2
    return pl.pallas_call(
        _k2_body,
        grid=(nblk,),
        in_specs=[pl.BlockSpec((AG, R2, NB), lambda r: (0, r, 0))],
        out_specs=[
            pl.BlockSpec((R2, D_SAE), lambda r: (r, 0)),
            pl.BlockSpec((R2, K), lambda r: (r, 0)),
            pl.BlockSpec((R2, K), lambda r: (r, 0)),
            pl.BlockSpec((R2, K * 16), lambda r: (r, 0)),
            pl.BlockSpec((1, 1), lambda r: (0, 0)),
        ],
        out_shape=[
            jax.ShapeDtypeStruct((N_TOK, D_SAE), jnp.float32),
            jax.ShapeDtypeStruct((N_TOK, K), jnp.float32),
            jax.ShapeDtypeStruct((N_TOK, K), jnp.int32),
            jax.ShapeDtypeStruct((N_TOK, K * 16), jnp.float32),
            jax.ShapeDtypeStruct((1, 1), jnp.float32),
        ],
        interpret=interpret,
    )(pre3d)


def _k3_body(idx_hbm, vals_hbm, wdec_hbm, bdec_hbm, out_hbm,
             idx_v, vals_v, bdec_v, bvals_v, rows_v, acc_v, sem0, sem1, osem):
    wid = lax.axis_index("s") * 2 + lax.axis_index("c")
    base = wid * TPW
    pltpu.sync_copy(idx_hbm.at[pl.ds(base * K, TPW * K)], idx_v)
    pltpu.sync_copy(bdec_hbm, bdec_v)
    sems = (sem0, sem1)

    def gather(t, buf, sem):
        start = pl.multiple_of(t * K, 8)
        pltpu.make_async_copy(
            wdec_hbm.at[idx_v.at[pl.ds(start, K)]], rows_v.at[buf],
            sem).start()

    gather(0, 0, sem0)

    def pair_body(p, _):
        for b in range(2):
            t = p * 2 + b
            nb = 1 - b

            @pl.when(t + 1 < TPW)
            def _():
                gather(t + 1, nb, sems[nb])

            pltpu.sync_copy(vals_hbm.at[base + t], vals_v)
            for k in range(K):
                bvals_v[k] = jnp.maximum(vals_v[pl.ds(k * 16, 16)], 0.0)
            pltpu.make_async_copy(
                wdec_hbm.at[idx_v.at[pl.ds(pl.multiple_of(t * K, 8), K)]],
                rows_v.at[b], sems[b]).wait()

            @pl.when(t >= 2)
            def _():
                pltpu.make_async_copy(
                    acc_v.at[b], out_hbm.at[base + t - 2], osem).wait()

            def cb(c, _):
                sl = pl.ds(pl.multiple_of(c * 16, 16), 16)

                def kb(kk, acc):
                    a0 = acc
                    for j in range(8):
                        a0 = a0 + bvals_v[kk * 8 + j] * rows_v[b, kk * 8 + j, sl]
                    return a0

                acc_v[b, sl] = lax.fori_loop(0, K // 8, kb, bdec_v[sl])
                return 0

            lax.fori_loop(0, D_MODEL // 16, cb, 0)
            pltpu.make_async_copy(
                acc_v.at[b], out_hbm.at[base + t], osem).start()
        return 0

    lax.fori_loop(0, TPW // 2, pair_body, 0)
    for b in range(2):
        pltpu.make_async_copy(
            acc_v.at[b], out_hbm.at[base + TPW - 2 + b], osem).wait()


def _decode_sc(idx2d, vals_b, W_dec, b_dec):
    mesh = plsc.VectorSubcoreMesh(core_axis_name="c", subcore_axis_name="s")
    f = functools.partial(
        pl.kernel,
        mesh=mesh,
        out_type=jax.ShapeDtypeStruct((N_TOK, D_MODEL), jnp.float32),
        scratch_types=[
            pltpu.VMEM((TPW * K,), jnp.int32),
            pltpu.VMEM((K * 16,), jnp.float32),
            pltpu.VMEM((D_MODEL,), jnp.float32),
            pltpu.VMEM((K, 16), jnp.float32),
            pltpu.VMEM((2, K, D_MODEL), jnp.float32),
            pltpu.VMEM((2, D_MODEL), jnp.float32),
            pltpu.SemaphoreType.DMA,
            pltpu.SemaphoreType.DMA,
            pltpu.SemaphoreType.DMA,
        ],
    )(_k3_body)
    return f(idx2d.reshape(-1), vals_b, W_dec, b_dec)


def _k4_body(x_ref, xr_ref, out_ref):
    rb = pl.program_id(0)

    @pl.when(rb == 0)
    def _():
        out_ref[...] = jnp.zeros((1, 1), jnp.float32)

    d = x_ref[...] - xr_ref[...]
    out_ref[...] += jnp.sum(d * d).reshape(1, 1)

    @pl.when(rb == pl.num_programs(0) - 1)
    def _():
        out_ref[...] = out_ref[...] / (N_TOK * D_MODEL)


def _loss_recon(x, x_rec, interpret=False):
    nblk = 8
    rb = N_TOK // nblk
    return pl.pallas_call(
        _k4_body,
        grid=(nblk,),
        in_specs=[
            pl.BlockSpec((rb, D_MODEL), lambda r: (r, 0)),
            pl.BlockSpec((rb, D_MODEL), lambda r: (r, 0)),
        ],
        out_specs=pl.BlockSpec((1, 1), lambda r: (0, 0)),
        out_shape=jax.ShapeDtypeStruct((1, 1), jnp.float32),
        interpret=interpret,
    )(x, x_rec)


def kernel(x, W_enc, b_enc, W_dec, b_dec):
    pre3d = _encode(x, W_enc, b_enc)
    acts, topk_values, topk_indices, vals_b, lsp = _topk_acts(pre3d)
    x_rec = _decode_sc(topk_indices, vals_b, W_dec, b_dec)
    lrec = _loss_recon(x, x_rec)
    return (x_rec, acts, topk_indices,
            lrec.reshape(()), lsp.reshape(()))
